# baseline (speedup 1.0000x reference)
import jax
import jax.numpy as jnp
from jax.experimental import pallas as pl
from jax.experimental.pallas import tpu as pltpu

_N = 10000
_F = 128
_BM_SUP = 2000
_BM = 400


def _support_kernel(x_ref, w_ref, b_ref, sup_ref):
    s = jnp.dot(x_ref[...], w_ref[...], preferred_element_type=jnp.float32)
    sup_ref[...] = s + b_ref[...]


def _agg_kernel(adj_ref, supk_ref, sup_ref, dn_ref, out_ref):
    neigh = jnp.dot(adj_ref[...], supk_ref[...],
                    preferred_element_type=jnp.float32)
    out_ref[...] = jnp.maximum(sup_ref[...] - dn_ref[...] * neigh, 0.0)


def kernel(x, adj_matrix, degree_norm, W, b):
    b2 = b.reshape(1, _F)
    sup = pl.pallas_call(
        _support_kernel,
        grid=(_N // _BM_SUP,),
        in_specs=[
            pl.BlockSpec((_BM_SUP, _F), lambda i: (i, 0)),
            pl.BlockSpec((_F, _F), lambda i: (0, 0)),
            pl.BlockSpec((1, _F), lambda i: (0, 0)),
        ],
        out_specs=pl.BlockSpec((_BM_SUP, _F), lambda i: (i, 0)),
        out_shape=jax.ShapeDtypeStruct((_N, _F), jnp.float32),
        compiler_params=pltpu.CompilerParams(
            dimension_semantics=("arbitrary",)),
    )(x, W, b2)

    out = pl.pallas_call(
        _agg_kernel,
        grid=(_N // _BM,),
        in_specs=[
            pl.BlockSpec((_BM, _N), lambda i: (i, 0)),
            pl.BlockSpec((_N, _F), lambda i: (0, 0)),
            pl.BlockSpec((_BM, _F), lambda i: (i, 0)),
            pl.BlockSpec((_BM, 1), lambda i: (i, 0)),
        ],
        out_specs=pl.BlockSpec((_BM, _F), lambda i: (i, 0)),
        out_shape=jax.ShapeDtypeStruct((_N, _F), jnp.float32),
        compiler_params=pltpu.CompilerParams(
            dimension_semantics=("arbitrary",)),
    )(adj_matrix, sup, sup, degree_norm)
    return out


# speedup vs baseline: 1.0728x; 1.0728x over previous
import jax
import jax.numpy as jnp
from jax.experimental import pallas as pl
from jax.experimental.pallas import tpu as pltpu

_N = 10000
_F = 128
_BM = 400


def _gdn_kernel(x_ref, w_ref, b_ref, adj_ref, dn_ref, out_ref, sup_ref):
    i = pl.program_id(0)

    @pl.when(i == 0)
    def _support():
        sup_ref[...] = jnp.dot(
            x_ref[...], w_ref[...], preferred_element_type=jnp.float32
        ) + b_ref[...]

    @pl.when(i > 0)
    def _aggregate():
        neigh = jnp.dot(adj_ref[...], sup_ref[...],
                        preferred_element_type=jnp.float32)
        sup_rows = sup_ref[pl.ds((i - 1) * _BM, _BM), :]
        out_ref[...] = jnp.maximum(sup_rows - dn_ref[...] * neigh, 0.0)


def kernel(x, adj_matrix, degree_norm, W, b):
    b2 = b.reshape(1, _F)
    num_i = _N // _BM

    def _clamped(i):
        return (jnp.maximum(i - 1, 0), 0)

    out = pl.pallas_call(
        _gdn_kernel,
        grid=(num_i + 1,),
        in_specs=[
            pl.BlockSpec((_N, _F), lambda i: (0, 0)),
            pl.BlockSpec((_F, _F), lambda i: (0, 0)),
            pl.BlockSpec((1, _F), lambda i: (0, 0)),
            pl.BlockSpec((_BM, _N), _clamped),
            pl.BlockSpec((_BM, 1), _clamped),
        ],
        out_specs=pl.BlockSpec((_BM, _F), _clamped),
        out_shape=jax.ShapeDtypeStruct((_N, _F), jnp.float32),
        scratch_shapes=[pltpu.VMEM((_N, _F), jnp.float32)],
        compiler_params=pltpu.CompilerParams(
            dimension_semantics=("arbitrary",)),
    )(x, W, b2, adj_matrix, degree_norm)
    return out


# speedup vs baseline: 1.0812x; 1.0078x over previous
import jax
import jax.numpy as jnp
from jax.experimental import pallas as pl
from jax.experimental.pallas import tpu as pltpu

_N = 10000
_F = 128
_BM = 200


def _gdn_kernel(x_ref, w_ref, b_ref, adj_ref, dn_ref, out_ref, sup_ref):
    i = pl.program_id(0)

    @pl.when(i == 0)
    def _support():
        sup_ref[...] = jnp.dot(
            x_ref[...], w_ref[...], preferred_element_type=jnp.float32
        ) + b_ref[...]

    @pl.when(i > 0)
    def _aggregate():
        neigh = jnp.dot(adj_ref[...], sup_ref[...],
                        preferred_element_type=jnp.float32)
        sup_rows = sup_ref[pl.ds((i - 1) * _BM, _BM), :]
        out_ref[...] = jnp.maximum(sup_rows - dn_ref[...] * neigh, 0.0)


def kernel(x, adj_matrix, degree_norm, W, b):
    b2 = b.reshape(1, _F)
    num_i = _N // _BM

    def _clamped(i):
        return (jnp.maximum(i - 1, 0), 0)

    out = pl.pallas_call(
        _gdn_kernel,
        grid=(num_i + 1,),
        in_specs=[
            pl.BlockSpec((_N, _F), lambda i: (0, 0)),
            pl.BlockSpec((_F, _F), lambda i: (0, 0)),
            pl.BlockSpec((1, _F), lambda i: (0, 0)),
            pl.BlockSpec((_BM, _N), _clamped),
            pl.BlockSpec((_BM, 1), _clamped),
        ],
        out_specs=pl.BlockSpec((_BM, _F), _clamped),
        out_shape=jax.ShapeDtypeStruct((_N, _F), jnp.float32),
        scratch_shapes=[pltpu.VMEM((_N, _F), jnp.float32)],
        compiler_params=pltpu.CompilerParams(
            dimension_semantics=("arbitrary",)),
    )(x, W, b2, adj_matrix, degree_norm)
    return out
